# Initial kernel scaffold; baseline (speedup 1.0000x reference)
import jax
import jax.numpy as jnp
from jax.experimental import pallas as pl


def kernel(x, cosine, label, w, b, w2, w3, b2, b3):
    raise NotImplementedError("write your pallas kernel here")


# speedup vs baseline: 7.5796x; 7.5796x over previous
import functools

import jax
import jax.numpy as jnp
from jax import lax
from jax.experimental import pallas as pl
from jax.experimental.pallas import tpu as pltpu

B = 1024
D = 128
V = 100000
K = 101
ROWS = 2 * B
R_BLK = 16


def _hs(t):
    return jnp.clip((t + 3.0) / 6.0, 0.0, 1.0)


def _stats_body(cos_ref, s_ref, ss_ref, t_ref):
    v = cos_ref[...]
    vi = lax.bitcast_convert_type(v, jnp.int32)

    lo = jnp.zeros((R_BLK, 1), jnp.int32)
    hi = jnp.full((R_BLK, 1), 0x40000000, jnp.int32)

    def step(_, carry):
        lo, hi = carry
        mid = (lo + hi) >> 1
        cnt = jnp.sum((vi >= mid).astype(jnp.int32), axis=1, keepdims=True)
        pred = cnt >= K
        return jnp.where(pred, mid, lo), jnp.where(pred, hi, mid)

    lo, hi = lax.fori_loop(0, 31, step, (lo, hi))
    thr = lax.bitcast_convert_type(lo, jnp.float32)

    dv = jnp.maximum(v - thr, 0.0)
    s_ref[...] = jnp.sum(dv, axis=1, keepdims=True)
    ss_ref[...] = jnp.sum(dv * dv, axis=1, keepdims=True)
    t_ref[...] = thr


def _loss_body(xp_ref, xa_ref, sp_ref, ssp_ref, tp_ref, sa_ref, ssa_ref,
               ta_ref, scal_ref, out_ref):
    w = scal_ref[0, 0]
    b = scal_ref[0, 1]
    w2 = scal_ref[0, 2]
    w3 = scal_ref[0, 3]
    b2 = scal_ref[0, 4]
    b3 = scal_ref[0, 5]

    xp = xp_ref[...]
    xa = xa_ref[...]
    eps = jnp.float32(1e-8)
    n_p = jnp.maximum(jnp.sqrt(jnp.sum(xp * xp, axis=1, keepdims=True)), eps)
    n_a = jnp.maximum(jnp.sqrt(jnp.sum(xa * xa, axis=1, keepdims=True)), eps)
    dot = lax.dot_general(xp, xa, (((1,), (1,)), ((), ())),
                          preferred_element_type=jnp.float32)
    out_dot = dot / (n_p * n_a.T)

    kf = jnp.float32(K)
    km1 = jnp.float32(K - 1)
    mean_a = ta_ref[...] + sa_ref[...] / kf
    var_a = jnp.maximum(ssa_ref[...] - sa_ref[...] * sa_ref[...] / kf, 0.0) / km1
    std_a = jnp.sqrt(var_a)
    mean_p = tp_ref[...] + sp_ref[...] / kf
    var_p = jnp.maximum(ssp_ref[...] - sp_ref[...] * sp_ref[...] / kf, 0.0) / km1
    std_p = jnp.sqrt(var_p)

    d1 = _hs(mean_a * w2 + w3).T
    s1 = _hs(std_a * b2 + b3).T
    d2 = _hs(mean_p * w2 + w3)
    s2 = _hs(std_p * b2 + b3)

    odn = 0.5 * ((out_dot - d1) / s1 + (out_dot - d2) / s2)
    cs = odn * w + b

    rmax = jnp.max(cs, axis=1, keepdims=True)
    lse = jnp.log(jnp.sum(jnp.exp(cs - rmax), axis=1, keepdims=True)) + rmax
    ii = lax.broadcasted_iota(jnp.int32, (B, B), 0)
    jj = lax.broadcasted_iota(jnp.int32, (B, B), 1)
    diag = jnp.sum(jnp.where(ii == jj, cs, 0.0), axis=1, keepdims=True)
    out_ref[0, 0] = jnp.mean(lse - diag)


@functools.partial(jax.jit, static_argnames=())
def kernel(x, cosine, label, w, b, w2, w3, b2, b3):
    del label
    rows = cosine.reshape(ROWS, V)

    s, ss, t = pl.pallas_call(
        _stats_body,
        grid=(ROWS // R_BLK,),
        in_specs=[pl.BlockSpec((R_BLK, V), lambda i: (i, 0))],
        out_specs=[pl.BlockSpec((R_BLK, 1), lambda i: (i, 0))] * 3,
        out_shape=[jax.ShapeDtypeStruct((ROWS, 1), jnp.float32)] * 3,
    )(rows)

    sp, sa = s[0::2], s[1::2]
    ssp, ssa = ss[0::2], ss[1::2]
    tp, ta = t[0::2], t[1::2]
    xp = x[:, 0, :]
    xa = x[:, 1, :]
    scal = jnp.stack([w, b, w2, w3, b2, b3]).reshape(1, 6).astype(jnp.float32)

    out = pl.pallas_call(
        _loss_body,
        in_specs=[pl.BlockSpec(memory_space=pltpu.VMEM)] * 8
        + [pl.BlockSpec(memory_space=pltpu.SMEM)],
        out_specs=pl.BlockSpec(memory_space=pltpu.SMEM),
        out_shape=jax.ShapeDtypeStruct((1, 1), jnp.float32),
    )(xp, xa, sp, ssp, tp, sa, ssa, ta, scal)
    return out[0, 0]
